# baseline (speedup 1.0000x reference)
import functools

import jax
import jax.numpy as jnp
from jax.experimental import pallas as pl
from jax.experimental.pallas import tpu as pltpu

NV = 8192
TAU = 2.0
BR = 256


@functools.lru_cache(maxsize=1)
def _gumbel_factor(n):
    g = jax.random.gumbel(jax.random.key(42), (n, NV), jnp.float32)
    return jax.device_put(jnp.exp(g * (1.0 / TAU)).astype(jnp.bfloat16))


def _vq_body(x_ref, cb_ref, w_ref, q_ref, p_ref):
    x = x_ref[...]
    cb = cb_ref[...]
    x2 = jnp.sum(x * x, axis=1, keepdims=True)
    c2 = jnp.sum(cb * cb, axis=1)[None, :]
    xc = jax.lax.dot_general(
        x, cb, (((1,), (1,)), ((), ())),
        preferred_element_type=jnp.float32)
    d2 = jnp.maximum(x2 + c2 - 2.0 * xc, 1e-12)
    w = w_ref[...].astype(jnp.float32)
    e = jnp.exp(jnp.sqrt(d2) * (-1.0 / TAU)) * w
    p = e * (1.0 / jnp.sum(e, axis=1, keepdims=True))
    p_ref[...] = p
    q_ref[...] = jnp.dot(p, cb, preferred_element_type=jnp.float32)


def kernel(x, codebook):
    b, t, d = x.shape
    n = b * t
    xf = x.reshape(n, d)
    w = _gumbel_factor(n)
    q, p = pl.pallas_call(
        _vq_body,
        grid=(n // BR,),
        in_specs=[
            pl.BlockSpec((BR, d), lambda i: (i, 0)),
            pl.BlockSpec((NV, d), lambda i: (0, 0)),
            pl.BlockSpec((BR, NV), lambda i: (i, 0)),
        ],
        out_specs=[
            pl.BlockSpec((BR, d), lambda i: (i, 0)),
            pl.BlockSpec((BR, NV), lambda i: (i, 0)),
        ],
        out_shape=[
            jax.ShapeDtypeStruct((n, d), jnp.float32),
            jax.ShapeDtypeStruct((n, NV), jnp.float32),
        ],
    )(xf, codebook, w)
    return q.reshape(b, t, d), p.reshape(b, t, NV)


# speedup vs baseline: 1.0232x; 1.0232x over previous
import jax
import jax.numpy as jnp
from jax.experimental import pallas as pl
from jax.experimental.pallas import tpu as pltpu

NV = 8192
TAU = 2.0
BR = 256
_TINY = 1.1754943508222875e-38


def _vq_body(x_ref, cb_ref, bits_ref, q_ref, p_ref):
    x = x_ref[...]
    cb = cb_ref[...]
    x2 = jnp.sum(x * x, axis=1, keepdims=True)
    c2 = jnp.sum(cb * cb, axis=1)[None, :]
    xc = jax.lax.dot_general(
        x, cb, (((1,), (1,)), ((), ())),
        preferred_element_type=jnp.float32)
    d2 = jnp.maximum(x2 + c2 - 2.0 * xc, 1e-12)
    ed = jnp.exp(jnp.sqrt(d2) * (-1.0 / TAU))
    fb = (bits_ref[...] >> jnp.uint32(9)) | jnp.uint32(0x3F800000)
    f = jax.lax.bitcast_convert_type(fb, jnp.float32) - 1.0
    u = jnp.maximum(f, _TINY)
    e = ed * jax.lax.rsqrt(-jnp.log(u))
    p = e * (1.0 / jnp.sum(e, axis=1, keepdims=True))
    p_ref[...] = p
    q_ref[...] = jnp.dot(p, cb, preferred_element_type=jnp.float32)


def kernel(x, codebook):
    b, t, d = x.shape
    n = b * t
    xf = x.reshape(n, d)
    bits = jax.random.bits(jax.random.key(42), (n, NV), jnp.uint32)
    q, p = pl.pallas_call(
        _vq_body,
        grid=(n // BR,),
        in_specs=[
            pl.BlockSpec((BR, d), lambda i: (i, 0)),
            pl.BlockSpec((NV, d), lambda i: (0, 0)),
            pl.BlockSpec((BR, NV), lambda i: (i, 0)),
        ],
        out_specs=[
            pl.BlockSpec((BR, d), lambda i: (i, 0)),
            pl.BlockSpec((BR, NV), lambda i: (i, 0)),
        ],
        out_shape=[
            jax.ShapeDtypeStruct((n, d), jnp.float32),
            jax.ShapeDtypeStruct((n, NV), jnp.float32),
        ],
    )(xf, codebook, bits)
    return q.reshape(b, t, d), p.reshape(b, t, NV)
